# baseline (speedup 1.0000x reference)
import jax
import jax.numpy as jnp
from jax import lax
from jax.experimental import pallas as pl
from jax.experimental.pallas import tpu as pltpu
from jax.experimental.pallas import tpu_sc as plsc

NC = 2
NS = 16
L = 16
NW = NC * NS

N_T = 128
T0, T1 = 300, 600
D = 256 * 9 * 16

N_SC = 8
SC_BASE = N_T - N_SC
VRS = 32
DSS = D // VRS
HQ = VRS // 4

N_TC = SC_BASE
ROWS = D // 128


def _sc_body(t_hbm, vg0_hbm, vg1_hbm, out_hbm,
             t_v, b0, b1, g0, g1, o0, o1):
    wid = lax.axis_index("s") * NC + lax.axis_index("c")
    k = wid // 4
    h = wid % 4
    lane = lax.broadcasted_iota(jnp.int32, (L,), 0)

    pltpu.sync_copy(t_hbm, t_v)
    tsv = jnp.broadcast_to(SC_BASE + k, (L,))
    tv = plsc.load_gather(t_v, [tsv])

    def chunk_state(tdim):
        inp = tv * float(tdim)
        li = jnp.minimum((inp + 1e-6).astype(jnp.int32), tdim - 1)
        ri = jnp.minimum(li + 1, tdim - 1)
        lif = li.astype(jnp.float32)
        wr = inp - lif
        wl = (lif + 1.0) - inp
        idx = jnp.where(lane < HQ,
                        li * VRS + h * HQ + lane,
                        ri * VRS + h * HQ + (lane - HQ))
        return idx, wl, wr

    idx0, wl0, wr0 = chunk_state(T0)
    idx1, wl1, wr1 = chunk_state(T1)

    gh0 = pltpu.async_copy(vg0_hbm.at[idx0], b0, g0)
    gh1 = pltpu.async_copy(vg1_hbm.at[idx1], b1, g1)

    ohs = []
    for gh, buf, wl, wr, g, osem in ((gh0, b0, wl0, wr0, 0, o0),
                                     (gh1, b1, wl1, wr1, 1, o1)):
        gh.wait()

        @plsc.parallel_loop(0, DSS, step=L, unroll=4)
        def _(off, buf=buf, wl=wl, wr=wr):
            for srow in range(HQ):
                vl = buf[srow, pl.ds(off, L)]
                vr = buf[srow + HQ, pl.ds(off, L)]
                buf[srow, pl.ds(off, L)] = wl * vl + wr * vr

        ohs.append(pltpu.async_copy(buf.at[pl.ds(0, HQ)],
                                    out_hbm.at[k, g, h], osem))
    for oh in ohs:
        oh.wait()


def _sc_run(t, vg0f, vg1f):
    mesh = plsc.VectorSubcoreMesh(core_axis_name="c", subcore_axis_name="s")
    run = pl.kernel(
        _sc_body,
        out_type=jax.ShapeDtypeStruct((N_SC, 2, 4, HQ, DSS), jnp.float32),
        mesh=mesh,
        compiler_params=pltpu.CompilerParams(needs_layout_passes=False),
        scratch_types=[
            pltpu.VMEM((N_T,), jnp.float32),
            pltpu.VMEM((2 * HQ, DSS), jnp.float32),
            pltpu.VMEM((2 * HQ, DSS), jnp.float32),
            pltpu.SemaphoreType.DMA,
            pltpu.SemaphoreType.DMA,
            pltpu.SemaphoreType.DMA,
            pltpu.SemaphoreType.DMA,
        ],
    )
    return run(t, vg0f, vg1f)


def _tc_body(li0_r, ri0_r, li1_r, ri1_r, t_r, l0, r0, l1, r1, out):
    i = pl.program_id(0)
    tval = t_r[i]

    inp0 = tval * float(T0)
    lif0 = li0_r[i].astype(jnp.float32)
    wr0 = inp0 - lif0
    wl0 = (lif0 + 1.0) - inp0
    out[0, 0] = wl0 * l0[0] + wr0 * r0[0]

    inp1 = tval * float(T1)
    lif1 = li1_r[i].astype(jnp.float32)
    wr1 = inp1 - lif1
    wl1 = (lif1 + 1.0) - inp1
    out[0, 1] = wl1 * l1[0] + wr1 * r1[0]


def _tc_run(t, vg0r, vg1r):
    def mk_idx(tdim):
        inp = t * tdim
        li = jnp.minimum(jnp.floor(inp + 1e-6).astype(jnp.int32), tdim - 1)
        ri = jnp.minimum(li + 1, tdim - 1)
        return li, ri

    li0, ri0 = mk_idx(T0)
    li1, ri1 = mk_idx(T1)

    spec = pltpu.PrefetchScalarGridSpec(
        num_scalar_prefetch=5,
        grid=(N_TC,),
        in_specs=[
            pl.BlockSpec((1, ROWS, 128),
                         lambda i, li0, ri0, li1, ri1, t: (li0[i], 0, 0)),
            pl.BlockSpec((1, ROWS, 128),
                         lambda i, li0, ri0, li1, ri1, t: (ri0[i], 0, 0)),
            pl.BlockSpec((1, ROWS, 128),
                         lambda i, li0, ri0, li1, ri1, t: (li1[i], 0, 0)),
            pl.BlockSpec((1, ROWS, 128),
                         lambda i, li0, ri0, li1, ri1, t: (ri1[i], 0, 0)),
        ],
        out_specs=pl.BlockSpec((1, 2, ROWS, 128),
                               lambda i, li0, ri0, li1, ri1, t: (i, 0, 0, 0)),
    )
    return pl.pallas_call(
        _tc_body,
        grid_spec=spec,
        out_shape=jax.ShapeDtypeStruct((N_T, 2, ROWS, 128), jnp.float32),
    )(li0, ri0, li1, ri1, t, vg0r, vg0r, vg1r, vg1r)


def kernel(t, vg0, vg1):
    vg0r = vg0.reshape(T0, ROWS, 128)
    vg1r = vg1.reshape(T1, ROWS, 128)
    vg0f = vg0.reshape(T0 * VRS, DSS)
    vg1f = vg1.reshape(T1 * VRS, DSS)

    tc_out = _tc_run(t, vg0r, vg1r)
    sc_out = _sc_run(t, vg0f, vg1f)
    sc_r = sc_out.reshape(N_SC, 2, ROWS, 128)
    full = lax.dynamic_update_slice(tc_out, sc_r, (SC_BASE, 0, 0, 0))
    return full.reshape(N_T, 2 * 256, 9, 16)


# speedup vs baseline: 1.1892x; 1.1892x over previous
import jax
import jax.numpy as jnp
from jax import lax
from jax.experimental import pallas as pl
from jax.experimental.pallas import tpu as pltpu
from jax.experimental.pallas import tpu_sc as plsc

NC = 2
NS = 16
L = 16
NW = NC * NS

N_T = 128
T0, T1 = 300, 600
D = 256 * 9 * 16
VR = 16
DS = D // VR
HALF = VR // 2
TASKS = N_T // NW
NBUF = 3
CHUNKS = 2 * TASKS * 2


def _sc_body(t_hbm, vg0_hbm, vg1_hbm, out_hbm,
             t_v, b0, b1, b2, g0, g1, g2, o0, o1, o2):
    wid = lax.axis_index("s") * NC + lax.axis_index("c")
    base = wid * TASKS
    lane = lax.broadcasted_iota(jnp.int32, (L,), 0)
    bufs = (b0, b1, b2)
    gsems = (g0, g1, g2)
    osems = (o0, o1, o2)

    pltpu.sync_copy(t_hbm, t_v)

    sched = []
    for g, vg_hbm, tdim in ((0, vg0_hbm, T0), (1, vg1_hbm, T1)):
        for k in range(TASKS):
            for h in range(2):
                sched.append((vg_hbm, tdim, k, h, g))

    def chunk_state(c):
        vg_hbm, tdim, k, h, g = sched[c]
        tsv = jnp.broadcast_to(base + k, (L,))
        tv = plsc.load_gather(t_v, [tsv])
        inp = tv * float(tdim)
        li = jnp.minimum((inp + 1e-6).astype(jnp.int32), tdim - 1)
        ri = jnp.minimum(li + 1, tdim - 1)
        lif = li.astype(jnp.float32)
        wr = inp - lif
        wl = (lif + 1.0) - inp
        idx = jnp.where(lane < HALF,
                        li * VR + h * HALF + lane,
                        ri * VR + h * HALF + (lane - HALF))
        return vg_hbm, idx, wl, wr

    gh = [None] * NBUF
    oh = [None] * NBUF
    vg_hbm, idx, wl0, wr0 = chunk_state(0)
    gh[0] = pltpu.async_copy(vg_hbm.at[idx], bufs[0], gsems[0])
    weights = [(wl0, wr0)]
    for c in range(CHUNKS):
        s = c % NBUF
        if c + 1 < CHUNKS:
            sn = (c + 1) % NBUF
            if oh[sn] is not None:
                oh[sn].wait()
            vg_hbm, idx, wl_n, wr_n = chunk_state(c + 1)
            gh[sn] = pltpu.async_copy(vg_hbm.at[idx], bufs[sn], gsems[sn])
            weights.append((wl_n, wr_n))
        gh[s].wait()
        wl, wr = weights[c]
        buf = bufs[s]

        @plsc.parallel_loop(0, DS, step=L, unroll=4)
        def _(off, buf=buf, wl=wl, wr=wr):
            for srow in range(HALF):
                vl = buf[srow, pl.ds(off, L)]
                vr = buf[srow + HALF, pl.ds(off, L)]
                buf[srow, pl.ds(off, L)] = wl * vl + wr * vr

        _, _, k, h, g = sched[c]
        oh[s] = pltpu.async_copy(buf.at[pl.ds(0, HALF)],
                                 out_hbm.at[base + k, g, h],
                                 osems[s])
    for s in range(NBUF):
        if oh[s] is not None:
            oh[s].wait()


def kernel(t, vg0, vg1):
    vg0f = vg0.reshape(T0 * VR, DS)
    vg1f = vg1.reshape(T1 * VR, DS)
    mesh = plsc.VectorSubcoreMesh(core_axis_name="c", subcore_axis_name="s")
    run = pl.kernel(
        _sc_body,
        out_type=jax.ShapeDtypeStruct((N_T, 2, 2, HALF, DS), jnp.float32),
        mesh=mesh,
        compiler_params=pltpu.CompilerParams(needs_layout_passes=False),
        scratch_types=[
            pltpu.VMEM((N_T,), jnp.float32),
            pltpu.VMEM((2 * HALF, DS), jnp.float32),
            pltpu.VMEM((2 * HALF, DS), jnp.float32),
            pltpu.VMEM((2 * HALF, DS), jnp.float32),
            pltpu.SemaphoreType.DMA,
            pltpu.SemaphoreType.DMA,
            pltpu.SemaphoreType.DMA,
            pltpu.SemaphoreType.DMA,
            pltpu.SemaphoreType.DMA,
            pltpu.SemaphoreType.DMA,
        ],
    )
    out = run(t, vg0f, vg1f)
    return out.reshape(N_T, 2 * 256, 9, 16)
